# baseline (speedup 1.0000x reference)
import jax
import jax.numpy as jnp
from jax import lax
from jax.experimental import pallas as pl
from jax.experimental.pallas import tpu as pltpu

_B, _F = 16384, 128
_CH = 512
_NCH = _B // _CH


def _transform(x, m1, m2, m3):
    xb = lax.bitcast_convert_type(x, jnp.int32)
    sbit = jnp.bitwise_and(xb, jnp.int32(-2147483648))
    ax = lax.bitcast_convert_type(
        jnp.bitwise_and(xb, jnp.int32(0x7FFFFFFF)), jnp.float32
    )
    l = lax.log(ax + 1.0)
    t1 = lax.bitcast_convert_type(
        jnp.bitwise_or(lax.bitcast_convert_type(l, jnp.int32), sbit), jnp.float32
    )
    s = ax * lax.rsqrt(ax + 1e-35)
    t2 = lax.bitcast_convert_type(
        jnp.bitwise_or(lax.bitcast_convert_type(s, jnp.int32), sbit), jnp.float32
    )
    out = jnp.where(m1, t1, x)
    out = jnp.where(m2, t2, out)
    return jnp.where(m3, x * x, out)


def _body(w_ref, x_hbm, o_hbm, xbuf, obuf, insem, outsem):
    m1 = w_ref[1:2, :] > 0.5
    m2 = w_ref[2:3, :] > 0.5
    m3 = w_ref[3:4, :] > 0.5

    def in_copy(k):
        return pltpu.make_async_copy(
            x_hbm.at[pl.ds(k * _CH, _CH), :], xbuf.at[k], insem.at[k]
        )

    def out_copy(k):
        return pltpu.make_async_copy(
            obuf.at[k], o_hbm.at[pl.ds(k * _CH, _CH), :], outsem.at[k]
        )

    for k in range(_NCH):
        in_copy(k).start()
    for k in range(_NCH):
        in_copy(k).wait()
        obuf[k] = _transform(xbuf[k], m1, m2, m3)
        out_copy(k).start()
    for k in range(_NCH):
        out_copy(k).wait()


@jax.jit
def _fused(X, wT):
    return pl.pallas_call(
        _body,
        in_specs=[
            pl.BlockSpec((8, _F), lambda: (0, 0)),
            pl.BlockSpec(memory_space=pltpu.MemorySpace.HBM),
        ],
        out_specs=pl.BlockSpec(memory_space=pltpu.MemorySpace.HBM),
        out_shape=jax.ShapeDtypeStruct(X.shape, X.dtype),
        scratch_shapes=[
            pltpu.VMEM((_NCH, _CH, _F), jnp.float32),
            pltpu.VMEM((_NCH, _CH, _F), jnp.float32),
            pltpu.SemaphoreType.DMA((_NCH,)),
            pltpu.SemaphoreType.DMA((_NCH,)),
        ],
    )(wT, X)


def kernel(X, tf_prob_logits, tf_prob_sample, is_fit, X_type):
    wT = jnp.zeros((8, _F), jnp.float32).at[0:4, :].set(tf_prob_sample.T)
    return _fused(X, wT)


# speedup vs baseline: 1.0455x; 1.0455x over previous
import jax
import jax.numpy as jnp
from jax import lax
from jax.experimental import pallas as pl
from jax.experimental.pallas import tpu as pltpu

_B, _F = 16384, 128
_CH = 1024
_NCH = _B // _CH


def _transform(x, m1, m2, m3):
    xb = lax.bitcast_convert_type(x, jnp.int32)
    sbit = jnp.bitwise_and(xb, jnp.int32(-2147483648))
    ax = lax.bitcast_convert_type(
        jnp.bitwise_and(xb, jnp.int32(0x7FFFFFFF)), jnp.float32
    )
    l = lax.log(ax + 1.0)
    s = ax * lax.rsqrt(ax + 1e-35)
    u = jnp.where(m1, l, s)
    su = lax.bitcast_convert_type(
        jnp.bitwise_or(lax.bitcast_convert_type(u, jnp.int32), sbit), jnp.float32
    )
    out = jnp.where(jnp.logical_or(m1, m2), su, x)
    return jnp.where(m3, x * x, out)


def _body(w_ref, x_hbm, o_hbm, xbuf, obuf, insem, outsem):
    m1 = w_ref[1:2, :] > 0.5
    m2 = w_ref[2:3, :] > 0.5
    m3 = w_ref[3:4, :] > 0.5

    def in_copy(k):
        return pltpu.make_async_copy(
            x_hbm.at[pl.ds(k * _CH, _CH), :], xbuf.at[k], insem.at[k]
        )

    def out_copy(k):
        return pltpu.make_async_copy(
            obuf.at[k], o_hbm.at[pl.ds(k * _CH, _CH), :], outsem.at[k]
        )

    for k in range(_NCH):
        in_copy(k).start()
    for k in range(_NCH):
        in_copy(k).wait()
        obuf[k] = _transform(xbuf[k], m1, m2, m3)
        out_copy(k).start()
    for k in range(_NCH):
        out_copy(k).wait()


@jax.jit
def _fused(X, wT):
    return pl.pallas_call(
        _body,
        in_specs=[
            pl.BlockSpec((8, _F), lambda: (0, 0)),
            pl.BlockSpec(memory_space=pltpu.MemorySpace.HBM),
        ],
        out_specs=pl.BlockSpec(memory_space=pltpu.MemorySpace.HBM),
        out_shape=jax.ShapeDtypeStruct(X.shape, X.dtype),
        scratch_shapes=[
            pltpu.VMEM((_NCH, _CH, _F), jnp.float32),
            pltpu.VMEM((_NCH, _CH, _F), jnp.float32),
            pltpu.SemaphoreType.DMA((_NCH,)),
            pltpu.SemaphoreType.DMA((_NCH,)),
        ],
    )(wT, X)


def kernel(X, tf_prob_logits, tf_prob_sample, is_fit, X_type):
    wT = jnp.zeros((8, _F), jnp.float32).at[0:4, :].set(tf_prob_sample.T)
    return _fused(X, wT)
